# Initial kernel scaffold; baseline (speedup 1.0000x reference)
import jax
import jax.numpy as jnp
from jax.experimental import pallas as pl


def kernel(pos, batch, W1, b1, W2, b2, W3, b3, W4, b4, W5, b5):
    raise NotImplementedError("write your pallas kernel here")


# speedup vs baseline: 8.4244x; 8.4244x over previous
import functools

import jax
import jax.numpy as jnp
from jax import lax
from jax.experimental import pallas as pl
from jax.experimental.pallas import tpu as pltpu
from jax.experimental.pallas import tpu_sc as plsc

_N = 4096
_K = 20
_G = 4

_NC = 2
_NS = 16
_NW = _NC * _NS
_CHUNK = 128


def _prep_body(x_ref, wa_ref, wj_ref, b_ref, a_ref, c_ref, sq_ref):
    x = x_ref[...]
    a_ref[...] = (
        jnp.dot(x, wa_ref[...], preferred_element_type=jnp.float32) + b_ref[...]
    )
    c_ref[...] = jnp.dot(x, wj_ref[...], preferred_element_type=jnp.float32)
    sq_ref[...] = jnp.sum(x * x, axis=1, keepdims=True)


def _prep(x, wa, wj, b):
    n, _ = x.shape
    d_out = wa.shape[1]
    d_pad = wj.shape[1]
    return pl.pallas_call(
        _prep_body,
        out_shape=(
            jax.ShapeDtypeStruct((n, d_out), jnp.float32),
            jax.ShapeDtypeStruct((n, d_pad), jnp.float32),
            jax.ShapeDtypeStruct((n, 1), jnp.float32),
        ),
    )(x, wa, wj, b.reshape(1, -1))


_TOPK_BLK = 256


def _topk_body(xb_ref, xf_ref, sqc_ref, sqr_ref, bc_ref, br_ref, idx_ref):
    g = lax.dot_general(
        xb_ref[...], xf_ref[...], (((1,), (1,)), ((), ())),
        preferred_element_type=jnp.float32,
    )
    d = sqc_ref[...] + sqr_ref[...] - 2.0 * g
    d = jnp.where(bc_ref[...] != br_ref[...], jnp.inf, d)
    iota = lax.broadcasted_iota(jnp.int32, d.shape, 1)
    for s in range(_K):
        v = jnp.min(d, axis=1, keepdims=True)
        cand = jnp.where(d == v, iota, _N)
        ix = jnp.min(cand, axis=1, keepdims=True)
        idx_ref[:, s : s + 1] = ix
        d = jnp.where(iota == ix, jnp.inf, d)


def _topk(x, sq, batch_col, batch_row):
    n, d_in = x.shape
    grid = n // _TOPK_BLK
    return pl.pallas_call(
        _topk_body,
        grid=(grid,),
        in_specs=[
            pl.BlockSpec((_TOPK_BLK, d_in), lambda i: (i, 0)),
            pl.BlockSpec((n, d_in), lambda i: (0, 0)),
            pl.BlockSpec((_TOPK_BLK, 1), lambda i: (i, 0)),
            pl.BlockSpec((1, n), lambda i: (0, 0)),
            pl.BlockSpec((_TOPK_BLK, 1), lambda i: (i, 0)),
            pl.BlockSpec((1, n), lambda i: (0, 0)),
        ],
        out_specs=pl.BlockSpec((_TOPK_BLK, _K), lambda i: (i, 0)),
        out_shape=jax.ShapeDtypeStruct((n, _K), jnp.int32),
    )(x, x, sq, sq.reshape(1, n), batch_col, batch_row)


def _sc_gather(table, idx_flat):
    b_total = idx_flat.shape[0]
    d = table.shape[1]
    b_per_w = b_total // _NW
    n_ch = b_per_w // _CHUNK
    mesh = plsc.VectorSubcoreMesh(
        core_axis_name="c", subcore_axis_name="s",
        num_cores=_NC, num_subcores=_NS,
    )

    @functools.partial(
        pl.kernel,
        out_type=jax.ShapeDtypeStruct((b_total, d), jnp.float32),
        mesh=mesh,
        scratch_types=[
            pltpu.VMEM((_CHUNK,), jnp.int32),
            pltpu.VMEM((_CHUNK, d), jnp.float32),
            pltpu.SemaphoreType.DMA,
        ],
    )
    def gather_kernel(table_hbm, idx_hbm, out_hbm, idx_c, rows_v, sem):
        wid = lax.axis_index("s") * _NC + lax.axis_index("c")
        base = wid * b_per_w

        def body(ch, carry):
            e0 = base + ch * _CHUNK
            pltpu.sync_copy(idx_hbm.at[pl.ds(e0, _CHUNK)], idx_c)
            pltpu.async_copy(table_hbm.at[idx_c], rows_v, sem).wait()
            pltpu.sync_copy(rows_v, out_hbm.at[pl.ds(e0, _CHUNK)])
            return carry

        lax.fori_loop(0, n_ch, body, 0)

    return gather_kernel(table, idx_flat)


_AGG_BLK = 256


def _aggr_body(d_out, g_ref, a_ref, o_ref):
    m = jnp.max(g_ref[...][:, :, :d_out], axis=1)
    o_ref[...] = jnp.maximum(a_ref[...] + m, 0.0)


def _aggr(g3, a):
    n, k, d_pad = g3.shape
    d_out = a.shape[1]
    grid = n // _AGG_BLK
    return pl.pallas_call(
        functools.partial(_aggr_body, d_out),
        grid=(grid,),
        in_specs=[
            pl.BlockSpec((_AGG_BLK, k, d_pad), lambda i: (i, 0, 0)),
            pl.BlockSpec((_AGG_BLK, d_out), lambda i: (i, 0)),
        ],
        out_specs=pl.BlockSpec((_AGG_BLK, d_out), lambda i: (i, 0)),
        out_shape=jax.ShapeDtypeStruct((n, d_out), jnp.float32),
    )(g3, a)


def _edge_conv(x, batch_col, batch_row, w, b):
    d_in = x.shape[1]
    d_out = w.shape[1]
    d_pad = -(-d_out // 128) * 128
    wi, wj = w[:d_in], w[d_in:]
    wj_p = jnp.pad(wj, ((0, 0), (0, d_pad - d_out)))
    a, c, sq = _prep(x, wi - wj, wj_p, b)
    idx = _topk(x, sq, batch_col, batch_row)
    g = _sc_gather(c, idx.reshape(-1))
    return _aggr(g.reshape(_N, _K, d_pad), a)


_FIN_BLK = 512


def _final_body(x_ref, w_ref, b_ref, bc_ref, br_ref, o_ref, sum_ref, cnt_ref):
    i = pl.program_id(0)
    f = w_ref.shape[1]

    @pl.when(i == 0)
    def _():
        o_ref[...] = jnp.full(o_ref.shape, -jnp.inf, jnp.float32)
        sum_ref[...] = jnp.zeros_like(sum_ref)
        cnt_ref[...] = jnp.zeros_like(cnt_ref)

    h = jnp.maximum(
        jnp.dot(x_ref[...], w_ref[...], preferred_element_type=jnp.float32)
        + b_ref[...],
        0.0,
    )
    bc = bc_ref[...]
    br = br_ref[...]
    gi = lax.broadcasted_iota(jnp.int32, (_G, br.shape[1]), 0)
    onehot = (gi == br).astype(jnp.float32)
    sum_ref[...] += jnp.dot(onehot, h, preferred_element_type=jnp.float32)
    cnt_ref[...] += jnp.sum(onehot, axis=1, keepdims=True)
    parts = [
        jnp.max(jnp.where(bc == g, h, -jnp.inf), axis=0, keepdims=True)
        for g in range(_G)
    ]
    pmax = jnp.concatenate(parts, axis=0)
    o_ref[:, :f] = jnp.maximum(o_ref[:, :f], pmax)

    @pl.when(i == pl.num_programs(0) - 1)
    def _():
        o_ref[:, f:] = sum_ref[...] / cnt_ref[:, :1]


def _final(xcat, w5, b5, batch_col, batch_row):
    n, d_in = xcat.shape
    f = w5.shape[1]
    grid = n // _FIN_BLK
    return pl.pallas_call(
        _final_body,
        grid=(grid,),
        in_specs=[
            pl.BlockSpec((_FIN_BLK, d_in), lambda i: (i, 0)),
            pl.BlockSpec((d_in, f), lambda i: (0, 0)),
            pl.BlockSpec((1, f), lambda i: (0, 0)),
            pl.BlockSpec((_FIN_BLK, 1), lambda i: (i, 0)),
            pl.BlockSpec((1, _FIN_BLK), lambda i: (0, i)),
        ],
        out_specs=pl.BlockSpec((_G, 2 * f), lambda i: (0, 0)),
        out_shape=jax.ShapeDtypeStruct((_G, 2 * f), jnp.float32),
        scratch_shapes=[
            pltpu.VMEM((_G, f), jnp.float32),
            pltpu.VMEM((_G, 1), jnp.float32),
        ],
    )(xcat, w5, b5.reshape(1, f), batch_col, batch_row)


def kernel(pos, batch, W1, b1, W2, b2, W3, b3, W4, b4, W5, b5):
    b32 = batch.astype(jnp.int32)
    bc = b32.reshape(-1, 1)
    br = b32.reshape(1, -1)
    x1 = _edge_conv(pos, bc, br, W1, b1)
    x2 = _edge_conv(x1, bc, br, W2, b2)
    x3 = _edge_conv(x2, bc, br, W3, b3)
    x4 = _edge_conv(x3, bc, br, W4, b4)
    xcat = jnp.concatenate([x1, x2, x3, x4], axis=1)
    return _final(xcat, W5, b5, bc, br)
